# baseline (device time: 32927 ns/iter reference)
import jax
import jax.numpy as jnp
from jax import lax
from jax.experimental import pallas as pl
from jax.experimental.pallas import tpu as pltpu

N_DEV = 8
LOG2_DEV = 3
N_TOK = 512
D_MODEL = 256
N_EXP = 16
EXP_PER_DEV = 2
H = 512


def kernel(x, router_W, route_idx, expert_W, shared_W):
    def body(
        x_ref,
        rw_ref,
        idx_ref,
        ew_ref,
        sw_ref,
        out_ref,
        send_buf,
        recv_buf,
        send_sems,
        recv_sems,
    ):
        my = lax.axis_index("i")

        barrier_sem = pltpu.get_barrier_semaphore()
        for r in range(LOG2_DEV):
            partner = my ^ (1 << r)
            pl.semaphore_signal(
                barrier_sem,
                inc=1,
                device_id=(partner,),
                device_id_type=pl.DeviceIdType.MESH,
            )
        pl.semaphore_wait(barrier_sem, LOG2_DEV)

        xf = x_ref[:, :]
        scores = jnp.dot(xf, rw_ref[:, :], preferred_element_type=jnp.float32)
        s_max = jnp.max(scores, axis=1, keepdims=True)
        p = jnp.exp(scores - s_max)
        probs = p / jnp.sum(p, axis=1, keepdims=True)

        idx = idx_ref[:, :]
        col = lax.broadcasted_iota(jnp.int32, (N_TOK, N_EXP), 1)

        acc = jnp.zeros((N_TOK, H), jnp.float32)
        for k in range(EXP_PER_DEV):
            e = my * EXP_PER_DEV + k
            sel = (col == e) & (idx == e)
            w = jnp.sum(jnp.where(sel, probs, 0.0), axis=1, keepdims=True)
            xw = (xf * w).astype(jnp.bfloat16)
            acc = acc + jnp.dot(
                xw,
                ew_ref[k].astype(jnp.bfloat16),
                preferred_element_type=jnp.float32,
            )

        for r in range(LOG2_DEV):
            partner = my ^ (1 << r)
            send_buf[:, :] = acc.astype(jnp.bfloat16)
            rdma = pltpu.make_async_remote_copy(
                src_ref=send_buf,
                dst_ref=recv_buf.at[r],
                send_sem=send_sems.at[r],
                recv_sem=recv_sems.at[r],
                device_id=(partner,),
                device_id_type=pl.DeviceIdType.MESH,
            )
            rdma.start()
            rdma.wait()
            acc = acc + recv_buf[r].astype(jnp.float32)

        shared = jnp.dot(
            xf.astype(jnp.bfloat16),
            sw_ref[:, :].astype(jnp.bfloat16),
            preferred_element_type=jnp.float32,
        )
        out_ref[:, :] = acc + shared

    return pl.pallas_call(
        body,
        out_shape=jax.ShapeDtypeStruct((N_TOK, H), jnp.float32),
        in_specs=[pl.BlockSpec(memory_space=pltpu.VMEM)] * 5,
        out_specs=pl.BlockSpec(memory_space=pltpu.VMEM),
        scratch_shapes=[
            pltpu.VMEM((N_TOK, H), jnp.bfloat16),
            pltpu.VMEM((LOG2_DEV, N_TOK, H), jnp.bfloat16),
            pltpu.SemaphoreType.DMA((LOG2_DEV,)),
            pltpu.SemaphoreType.DMA((LOG2_DEV,)),
        ],
        compiler_params=pltpu.CompilerParams(collective_id=0),
    )(x, router_W, route_idx, expert_W, shared_W)


# device time: 32239 ns/iter; 1.0213x vs baseline; 1.0213x over previous
import jax
import jax.numpy as jnp
from jax import lax
from jax.experimental import pallas as pl
from jax.experimental.pallas import tpu as pltpu

N_DEV = 8
LOG2_DEV = 3
N_TOK = 512
D_MODEL = 256
N_EXP = 16
EXP_PER_DEV = 2
H = 512


def kernel(x, router_W, route_idx, expert_W, shared_W):
    def body(
        x_ref,
        rw_ref,
        idx_ref,
        ew_ref,
        sw_ref,
        out_ref,
        acc_ref,
        recv_buf,
        send_sems,
        recv_sems,
    ):
        my = lax.axis_index("i")

        barrier_sem = pltpu.get_barrier_semaphore()
        for r in range(LOG2_DEV):
            partner = my ^ (1 << r)
            pl.semaphore_signal(
                barrier_sem,
                inc=1,
                device_id=(partner,),
                device_id_type=pl.DeviceIdType.MESH,
            )
        pl.semaphore_wait(barrier_sem, LOG2_DEV)

        xf = x_ref[:, :]
        scores = jnp.dot(xf, rw_ref[:, :], preferred_element_type=jnp.float32)
        s_max = jnp.max(scores, axis=1, keepdims=True)
        p = jnp.exp(scores - s_max)
        probs = p / jnp.sum(p, axis=1, keepdims=True)

        idx = idx_ref[:, :]
        col = lax.broadcasted_iota(jnp.int32, (N_TOK, N_EXP), 1)

        acc = jnp.zeros((N_TOK, H), jnp.float32)
        for k in range(EXP_PER_DEV):
            e = my * EXP_PER_DEV + k
            sel = (col == e) & (idx == e)
            w = jnp.sum(jnp.where(sel, probs, 0.0), axis=1, keepdims=True)
            xw = (xf * w).astype(jnp.bfloat16)
            acc = acc + jnp.dot(
                xw,
                ew_ref[k].astype(jnp.bfloat16),
                preferred_element_type=jnp.float32,
            )
        acc_ref[:, :] = acc.astype(jnp.bfloat16)

        shared = None
        start = my - my
        for r in range(LOG2_DEV):
            L = N_TOK >> (r + 1)
            bit = (my >> r) & 1
            partner = my ^ (1 << r)
            send_start = start + (1 - bit) * L
            keep_start = start + bit * L
            rdma = pltpu.make_async_remote_copy(
                src_ref=acc_ref.at[pl.ds(send_start, L)],
                dst_ref=recv_buf.at[r, pl.ds(0, L)],
                send_sem=send_sems.at[r],
                recv_sem=recv_sems.at[r],
                device_id=(partner,),
                device_id_type=pl.DeviceIdType.MESH,
            )
            rdma.start()
            if r == 0:
                shared = jnp.dot(
                    xf.astype(jnp.bfloat16),
                    sw_ref[:, :].astype(jnp.bfloat16),
                    preferred_element_type=jnp.float32,
                )
            rdma.wait()
            acc_ref[pl.ds(keep_start, L)] = (
                acc_ref[pl.ds(keep_start, L)] + recv_buf[r, pl.ds(0, L)]
            )
            start = keep_start

        for r in reversed(range(LOG2_DEV)):
            L = N_TOK >> (r + 1)
            partner = my ^ (1 << r)
            rdma = pltpu.make_async_remote_copy(
                src_ref=acc_ref.at[pl.ds(start, L)],
                dst_ref=acc_ref.at[pl.ds(start, L)],
                send_sem=send_sems.at[LOG2_DEV + r],
                recv_sem=recv_sems.at[LOG2_DEV + r],
                device_id=(partner,),
                device_id_type=pl.DeviceIdType.MESH,
            )
            rdma.start()
            rdma.wait()
            start = start - ((my >> r) & 1) * L

        out_ref[:, :] = acc_ref[:, :].astype(jnp.float32) + shared

    return pl.pallas_call(
        body,
        out_shape=jax.ShapeDtypeStruct((N_TOK, H), jnp.float32),
        in_specs=[pl.BlockSpec(memory_space=pltpu.VMEM)] * 5,
        out_specs=pl.BlockSpec(memory_space=pltpu.VMEM),
        scratch_shapes=[
            pltpu.VMEM((N_TOK, H), jnp.bfloat16),
            pltpu.VMEM((LOG2_DEV, N_TOK // 2, H), jnp.bfloat16),
            pltpu.SemaphoreType.DMA((2 * LOG2_DEV,)),
            pltpu.SemaphoreType.DMA((2 * LOG2_DEV,)),
        ],
        compiler_params=pltpu.CompilerParams(collective_id=0),
    )(x, router_W, route_idx, expert_W, shared_W)
